# Initial kernel scaffold; baseline (speedup 1.0000x reference)
import jax
import jax.numpy as jnp
from jax.experimental import pallas as pl


def kernel(users, items, user_emb, item_emb, edge_index, edge_weight):
    raise NotImplementedError("write your pallas kernel here")


# speedup vs baseline: 2.3069x; 2.3069x over previous
import functools

import jax
import jax.numpy as jnp
from jax import lax
from jax.experimental import pallas as pl
from jax.experimental.pallas import tpu as pltpu
from jax.experimental.pallas import tpu_sc as plsc

USER_N = 50000
NN = 100000
DIM = 32
E = 1600000
B = 4096
NC = 2
NS = 16
QTR = 25000
ACC_ROWS = 25088
TRASH = QTR
EP = 1638400
ROWS_PER_TILE = EP // 128 // NS
CHUNK_IR = 8
CHUNK_E = CHUNK_IR * 128
N_CHUNKS = ROWS_PER_TILE // CHUNK_IR
ZROWS = 784
STRIPE = 1568
WR_LAST = QTR - (NS - 1) * STRIPE

_MESH = plsc.VectorSubcoreMesh(core_axis_name="c", subcore_axis_name="s")
_CPARAMS = pltpu.CompilerParams(use_tc_tiling_on_sc=False)
_CPARAMS_NL = pltpu.CompilerParams(use_tc_tiling_on_sc=False,
                                   needs_layout_passes=False)


def _layer_body(x_hbm, src_hbm, dst_hbm, w_hbm, out_hbm,
                idx_v, dst_v, w_v, rows_v, zeros_v, acc_sh, gsem):
    c = lax.axis_index("c")
    s = lax.axis_index("s")

    z16 = jnp.zeros((16,), jnp.float32)

    def z_body(q, _):
        r = q // 2
        col = (q % 2) * 16
        zeros_v[r, pl.ds(col, 16)] = z16
        return 0

    lax.fori_loop(0, ZROWS * 2, z_body, 0, unroll=8)

    tile_row0 = s * ROWS_PER_TILE
    zbase = s * STRIPE

    for p in range(2):
        for k in range(2):
            pltpu.sync_copy(zeros_v,
                            acc_sh.at[pl.ds(zbase + k * ZROWS, ZROWS)])
        plsc.subcore_barrier()

        cbase = p * (NC * QTR) + c * QTR

        def chunk_body(ci, _):
            r0 = tile_row0 + ci * CHUNK_IR
            pltpu.sync_copy(src_hbm.at[pl.ds(r0, CHUNK_IR)], idx_v)
            pltpu.sync_copy(dst_hbm.at[pl.ds(r0, CHUNK_IR)], dst_v)
            pltpu.sync_copy(w_hbm.at[pl.ds(r0, CHUNK_IR)], w_v)
            cps = [pltpu.async_copy(x_hbm.at[idx_v.at[j]],
                                    rows_v.at[pl.ds(j * 128, 128)], gsem)
                   for j in range(CHUNK_IR)]
            for cp in cps:
                cp.wait()

            def e_body(gi, _):
                r = gi // 8
                col = (gi % 8) * 16
                w16 = w_v[r, pl.ds(col, 16)]
                for k in range(16):
                    j = gi * 16 + k
                    wk = w16[k]
                    rows_v[j, pl.ds(0, 16)] = rows_v[j, pl.ds(0, 16)] * wk
                    rows_v[j, pl.ds(16, 16)] = rows_v[j, pl.ds(16, 16)] * wk
                return 0

            lax.fori_loop(0, CHUNK_E // 16, e_body, 0)

            def d_body(q, _):
                r = q // 8
                col = (q % 8) * 16
                d = dst_v[r, pl.ds(col, 16)]
                ld = d - cbase
                ok = (ld >= 0) & (ld < QTR)
                dst_v[r, pl.ds(col, 16)] = jnp.where(ok, ld, TRASH)
                return 0

            lax.fori_loop(0, CHUNK_IR * 8, d_body, 0, unroll=8)

            for j in range(CHUNK_IR):
                pltpu.sync_copy(rows_v.at[pl.ds(j * 128, 128)],
                                acc_sh.at[dst_v.at[j]], add=True)
            return 0

        lax.fori_loop(0, N_CHUNKS, chunk_body, 0)
        plsc.subcore_barrier()

        @pl.when(s < NS - 1)
        def _():
            pltpu.sync_copy(acc_sh.at[pl.ds(s * STRIPE, STRIPE)],
                            out_hbm.at[pl.ds(cbase + s * STRIPE, STRIPE)])

        @pl.when(s == NS - 1)
        def _():
            pltpu.sync_copy(
                acc_sh.at[pl.ds((NS - 1) * STRIPE, WR_LAST)],
                out_hbm.at[pl.ds(cbase + (NS - 1) * STRIPE, WR_LAST)])


_layer = functools.partial(
    pl.kernel,
    out_type=jax.ShapeDtypeStruct((NN, DIM), jnp.float32),
    mesh=_MESH,
    compiler_params=_CPARAMS,
    scratch_types=[
        pltpu.VMEM((CHUNK_IR, 128), jnp.int32),
        pltpu.VMEM((CHUNK_IR, 128), jnp.int32),
        pltpu.VMEM((CHUNK_IR, 128), jnp.float32),
        pltpu.VMEM((CHUNK_E, DIM), jnp.float32),
        pltpu.VMEM((ZROWS, DIM), jnp.float32),
        pltpu.VMEM_SHARED((ACC_ROWS, DIM), jnp.float32),
        pltpu.SemaphoreType.DMA,
    ],
)(_layer_body)


def _final_body(x0, x1, x2, x3, u_hbm, i_hbm, out_hbm,
                uidx, iidx, uacc, vacc, tmp, g_v, gsem):
    c = lax.axis_index("c")
    s = lax.axis_index("s")
    wid = s * NC + c

    pltpu.sync_copy(u_hbm.at[pl.ds(wid * 128, 128)], uidx)
    pltpu.sync_copy(i_hbm.at[pl.ds(wid * 128, 128)], iidx)

    def acc_into(dst_ref, first, tabs, idx_ref):
        pltpu.async_copy(first.at[idx_ref], dst_ref, gsem).wait()
        for t in tabs:
            pltpu.async_copy(t.at[idx_ref], tmp, gsem).wait()

            def a_body(q, _):
                r = q // 2
                col = (q % 2) * 16
                dst_ref[r, pl.ds(col, 16)] = (dst_ref[r, pl.ds(col, 16)]
                                              + tmp[r, pl.ds(col, 16)])
                return 0

            lax.fori_loop(0, 256, a_body, 0, unroll=8)

    acc_into(uacc, x0, (x1, x2, x3), uidx)
    acc_into(vacc, x0, (x1, x2, x3), iidx)

    iota16 = lax.iota(jnp.int32, 16)

    def p_body(gi, _):
        g16 = jnp.zeros((16,), jnp.float32)
        for k in range(16):
            j = gi * 16 + k
            u0 = uacc[j, pl.ds(0, 16)]
            u1 = uacc[j, pl.ds(16, 16)]
            v0 = vacc[j, pl.ds(0, 16)]
            v1 = vacc[j, pl.ds(16, 16)]
            dot = jnp.sum(u0 * v0 + u1 * v1)
            g16 = g16 + jnp.where(iota16 == k, dot, 0.0)
        z = g16 * 0.0625
        g_v[pl.ds(gi * 16, 16)] = 1.0 / (1.0 + jnp.exp(-z))
        return 0

    lax.fori_loop(0, 8, p_body, 0)
    pltpu.sync_copy(g_v, out_hbm.at[pl.ds(wid * 128, 128)])


_final = functools.partial(
    pl.kernel,
    out_type=jax.ShapeDtypeStruct((B,), jnp.float32),
    mesh=_MESH,
    compiler_params=_CPARAMS_NL,
    scratch_types=[
        pltpu.VMEM((128,), jnp.int32),
        pltpu.VMEM((128,), jnp.int32),
        pltpu.VMEM((128, DIM), jnp.float32),
        pltpu.VMEM((128, DIM), jnp.float32),
        pltpu.VMEM((128, DIM), jnp.float32),
        pltpu.VMEM((128,), jnp.float32),
        pltpu.SemaphoreType.DMA,
    ],
)(_final_body)


def kernel(users, items, user_emb, item_emb, edge_index, edge_weight):
    x0 = jnp.concatenate([user_emb, item_emb], axis=0)
    src = edge_index[0].astype(jnp.int32)
    dst = edge_index[1].astype(jnp.int32)
    w = edge_weight.astype(jnp.float32)
    pad = EP - E
    src = jnp.concatenate([src, jnp.zeros((pad,), jnp.int32)]).reshape(EP // 128, 128)
    dst = jnp.concatenate([dst, jnp.full((pad,), NN, jnp.int32)]).reshape(EP // 128, 128)
    w = jnp.concatenate([w, jnp.zeros((pad,), jnp.float32)]).reshape(EP // 128, 128)

    x1 = _layer(x0, src, dst, w)
    x2 = _layer(x1, src, dst, w)
    x3 = _layer(x2, src, dst, w)

    u1d = users.astype(jnp.int32)
    i1d = items.astype(jnp.int32) + USER_N
    return _final(x0, x1, x2, x3, u1d, i1d)


# speedup vs baseline: 4.8587x; 2.1062x over previous
import functools

import jax
import jax.numpy as jnp
from jax import lax
from jax.experimental import pallas as pl
from jax.experimental.pallas import tpu as pltpu
from jax.experimental.pallas import tpu_sc as plsc

USER_N = 50000
NN = 100000
DIM = 32
E = 1600000
B = 4096
NC = 2
NS = 16
NT = NC * NS
QTR = 25000
NQ = 4
ACC_ROWS = 25088
TRASH = QTR
EP = 1638400
IR_PER_TILE = EP // 128 // NT
SCAN_IR = 8
N_SCAN = IR_PER_TILE // SCAN_IR
CHUNK_E = 1024
PAD_G = CHUNK_E // 16
RCAP_E = 103424
PTOT = NQ * NT * RCAP_E
ZROWS = 784
STRIPE = 1568
WR_LAST = QTR - (NS - 1) * STRIPE

_MESH = plsc.VectorSubcoreMesh(core_axis_name="c", subcore_axis_name="s")
_CPARAMS = pltpu.CompilerParams(use_tc_tiling_on_sc=False)
_CPARAMS_NL = pltpu.CompilerParams(use_tc_tiling_on_sc=False,
                                   needs_layout_passes=False)


def _partition_body(src_hbm, dst_hbm, w_hbm, psrc, pdst, pw, cnts,
                    cs_v, cd_v, cw_v, st_s, st_d, st_w, cnt_v):
    c = lax.axis_index("c")
    s = lax.axis_index("s")
    wid = s * NC + c
    iota16 = lax.iota(jnp.int32, 16)

    def flush(bt, nf):
        base = (bt * NT + wid) * RCAP_E + nf * CHUNK_E
        pltpu.sync_copy(st_s.at[bt].at[pl.ds(0, CHUNK_E)],
                        psrc.at[pl.ds(base, CHUNK_E)])
        pltpu.sync_copy(st_d.at[bt].at[pl.ds(0, CHUNK_E)],
                        pdst.at[pl.ds(base, CHUNK_E)])
        pltpu.sync_copy(st_w.at[bt].at[pl.ds(0, CHUNK_E)],
                        pw.at[pl.ds(base, CHUNK_E)])

    def group_step(carry, d, sg, wg):
        fills, nfs = carry
        bq = ((d >= QTR).astype(jnp.int32)
              + (d >= 2 * QTR).astype(jnp.int32)
              + (d >= 3 * QTR).astype(jnp.int32))
        new_fills, new_nfs = [], []
        for bt in range(NQ):
            mask = bq == bt
            cnt = plsc.all_reduce_population_count(mask)[0]
            fill = fills[bt]
            nf = nfs[bt]
            ld = d - bt * QTR
            plsc.store_compressed(st_s.at[bt].at[pl.ds(fill, 16)], sg,
                                  mask=mask)
            plsc.store_compressed(st_d.at[bt].at[pl.ds(fill, 16)], ld,
                                  mask=mask)
            plsc.store_compressed(st_w.at[bt].at[pl.ds(fill, 16)], wg,
                                  mask=mask)
            fill = fill + cnt
            do_flush = fill >= CHUNK_E

            @pl.when(do_flush)
            def _():
                flush(bt, nf)
                st_s.at[bt][pl.ds(0, 16)] = st_s.at[bt][pl.ds(CHUNK_E, 16)]
                st_d.at[bt][pl.ds(0, 16)] = st_d.at[bt][pl.ds(CHUNK_E, 16)]
                st_w.at[bt][pl.ds(0, 16)] = st_w.at[bt][pl.ds(CHUNK_E, 16)]

            new_fills.append(jnp.where(do_flush, fill - CHUNK_E, fill))
            new_nfs.append(jnp.where(do_flush, nf + 1, nf))
        return (tuple(new_fills), tuple(new_nfs))

    def chunk_body(ci, carry):
        r0 = wid * IR_PER_TILE + ci * SCAN_IR
        pltpu.sync_copy(src_hbm.at[pl.ds(r0, SCAN_IR)], cs_v)
        pltpu.sync_copy(dst_hbm.at[pl.ds(r0, SCAN_IR)], cd_v)
        pltpu.sync_copy(w_hbm.at[pl.ds(r0, SCAN_IR)], cw_v)

        def g_body(gi, carry):
            r = gi // 8
            col = (gi % 8) * 16
            d = cd_v[r, pl.ds(col, 16)]
            sg = cs_v[r, pl.ds(col, 16)]
            wg = cw_v[r, pl.ds(col, 16)]
            return group_step(carry, d, sg, wg)

        return lax.fori_loop(0, SCAN_IR * 8, g_body, carry)

    zero = jnp.int32(0)
    fills, nfs = lax.fori_loop(
        0, N_SCAN, chunk_body,
        ((zero, zero, zero, zero), (zero, zero, zero, zero)))

    t_s = jnp.zeros((16,), jnp.int32)
    t_d = jnp.full((16,), TRASH, jnp.int32)
    t_w = jnp.zeros((16,), jnp.float32)
    cv = jnp.zeros((16,), jnp.int32)
    for bt in range(NQ):
        fill = fills[bt]
        nf = nfs[bt]

        def pad_k(k, _):
            off = fill + 16 * k
            m = (iota16 * 0 + off) < CHUNK_E
            offc = jnp.minimum(off, CHUNK_E)
            plsc.store_compressed(st_s.at[bt].at[pl.ds(offc, 16)], t_s,
                                  mask=m)
            plsc.store_compressed(st_d.at[bt].at[pl.ds(offc, 16)], t_d,
                                  mask=m)
            plsc.store_compressed(st_w.at[bt].at[pl.ds(offc, 16)], t_w,
                                  mask=m)
            return 0

        lax.fori_loop(0, PAD_G, pad_k, 0)

        @pl.when(fill > 0)
        def _():
            flush(bt, nf)

        nf = jnp.where(fill > 0, nf + 1, nf)
        cv = cv + jnp.where(iota16 == bt, nf * CHUNK_E, 0)

    cnt_v[...] = cv
    pltpu.sync_copy(cnt_v, cnts.at[pl.ds(wid * 16, 16)])


_partition = functools.partial(
    pl.kernel,
    out_type=(
        jax.ShapeDtypeStruct((PTOT,), jnp.int32),
        jax.ShapeDtypeStruct((PTOT,), jnp.int32),
        jax.ShapeDtypeStruct((PTOT,), jnp.float32),
        jax.ShapeDtypeStruct((NT * 16,), jnp.int32),
    ),
    mesh=_MESH,
    compiler_params=_CPARAMS_NL,
    scratch_types=[
        pltpu.VMEM((SCAN_IR, 128), jnp.int32),
        pltpu.VMEM((SCAN_IR, 128), jnp.int32),
        pltpu.VMEM((SCAN_IR, 128), jnp.float32),
        pltpu.VMEM((NQ, CHUNK_E + 16), jnp.int32),
        pltpu.VMEM((NQ, CHUNK_E + 16), jnp.int32),
        pltpu.VMEM((NQ, CHUNK_E + 16), jnp.float32),
        pltpu.VMEM((16,), jnp.int32),
    ],
)(_partition_body)


def _layer_body(x_hbm, psrc, pdst, pw, cnts, out_hbm,
                idx_v, pd_v, w_v, rows_v, zeros_v, cnt_v, acc_sh, gsem):
    c = lax.axis_index("c")
    s = lax.axis_index("s")
    iota16 = lax.iota(jnp.int32, 16)

    z16 = jnp.zeros((16,), jnp.float32)

    def z_body(q, _):
        r = q // 2
        col = (q % 2) * 16
        zeros_v[r, pl.ds(col, 16)] = z16
        return 0

    lax.fori_loop(0, ZROWS * 2, z_body, 0, unroll=8)
    zbase = s * STRIPE

    for p in range(2):
        for k in range(2):
            pltpu.sync_copy(zeros_v,
                            acc_sh.at[pl.ds(zbase + k * ZROWS, ZROWS)])
        plsc.subcore_barrier()

        q = p * NC + c
        cbase = q * QTR

        def chunk_body(ci, rbase):
            off = rbase + ci * CHUNK_E
            pltpu.sync_copy(psrc.at[pl.ds(off, CHUNK_E)], idx_v)
            pltpu.sync_copy(pdst.at[pl.ds(off, CHUNK_E)], pd_v)
            pltpu.sync_copy(pw.at[pl.ds(off, CHUNK_E)], w_v)
            cps = [pltpu.async_copy(
                       x_hbm.at[idx_v.at[pl.ds(j * 128, 128)]],
                       rows_v.at[pl.ds(j * 128, 128)], gsem)
                   for j in range(CHUNK_E // 128)]
            for cp in cps:
                cp.wait()

            def e_body(gi, _):
                w16 = w_v[pl.ds(gi * 16, 16)]
                for k in range(16):
                    j = gi * 16 + k
                    wk = w16[k]
                    rows_v[j, pl.ds(0, 16)] = rows_v[j, pl.ds(0, 16)] * wk
                    rows_v[j, pl.ds(16, 16)] = rows_v[j, pl.ds(16, 16)] * wk
                return 0

            lax.fori_loop(0, CHUNK_E // 16, e_body, 0)

            for j in range(CHUNK_E // 128):
                pltpu.sync_copy(rows_v.at[pl.ds(j * 128, 128)],
                                acc_sh.at[pd_v.at[pl.ds(j * 128, 128)]],
                                add=True)
            return rbase

        for rg in range(2):
            pt = s * 2 + rg
            pltpu.sync_copy(cnts.at[pl.ds(pt * 16, 16)], cnt_v)
            cvec = cnt_v[...]
            n_edges = jnp.sum(jnp.where(iota16 == q, cvec, 0))
            n_chunks = n_edges // CHUNK_E
            rbase = (q * NT + pt) * RCAP_E
            lax.fori_loop(0, n_chunks, chunk_body, rbase)

        plsc.subcore_barrier()

        @pl.when(s < NS - 1)
        def _():
            pltpu.sync_copy(acc_sh.at[pl.ds(s * STRIPE, STRIPE)],
                            out_hbm.at[pl.ds(cbase + s * STRIPE, STRIPE)])

        @pl.when(s == NS - 1)
        def _():
            pltpu.sync_copy(
                acc_sh.at[pl.ds((NS - 1) * STRIPE, WR_LAST)],
                out_hbm.at[pl.ds(cbase + (NS - 1) * STRIPE, WR_LAST)])


_layer = functools.partial(
    pl.kernel,
    out_type=jax.ShapeDtypeStruct((NN, DIM), jnp.float32),
    mesh=_MESH,
    compiler_params=_CPARAMS_NL,
    scratch_types=[
        pltpu.VMEM((CHUNK_E,), jnp.int32),
        pltpu.VMEM((CHUNK_E,), jnp.int32),
        pltpu.VMEM((CHUNK_E,), jnp.float32),
        pltpu.VMEM((CHUNK_E, DIM), jnp.float32),
        pltpu.VMEM((ZROWS, DIM), jnp.float32),
        pltpu.VMEM((16,), jnp.int32),
        pltpu.VMEM_SHARED((ACC_ROWS, DIM), jnp.float32),
        pltpu.SemaphoreType.DMA,
    ],
)(_layer_body)


def _final_body(x0, x1, x2, x3, u_hbm, i_hbm, out_hbm,
                uidx, iidx, uacc, vacc, tmp, g_v, gsem):
    c = lax.axis_index("c")
    s = lax.axis_index("s")
    wid = s * NC + c

    pltpu.sync_copy(u_hbm.at[pl.ds(wid * 128, 128)], uidx)
    pltpu.sync_copy(i_hbm.at[pl.ds(wid * 128, 128)], iidx)

    def acc_into(dst_ref, first, tabs, idx_ref):
        pltpu.async_copy(first.at[idx_ref], dst_ref, gsem).wait()
        for t in tabs:
            pltpu.async_copy(t.at[idx_ref], tmp, gsem).wait()

            def a_body(qq, _):
                r = qq // 2
                col = (qq % 2) * 16
                dst_ref[r, pl.ds(col, 16)] = (dst_ref[r, pl.ds(col, 16)]
                                              + tmp[r, pl.ds(col, 16)])
                return 0

            lax.fori_loop(0, 256, a_body, 0, unroll=8)

    acc_into(uacc, x0, (x1, x2, x3), uidx)
    acc_into(vacc, x0, (x1, x2, x3), iidx)

    iota16 = lax.iota(jnp.int32, 16)

    def p_body(gi, _):
        g16 = jnp.zeros((16,), jnp.float32)
        for k in range(16):
            j = gi * 16 + k
            u0 = uacc[j, pl.ds(0, 16)]
            u1 = uacc[j, pl.ds(16, 16)]
            v0 = vacc[j, pl.ds(0, 16)]
            v1 = vacc[j, pl.ds(16, 16)]
            dot = jnp.sum(u0 * v0 + u1 * v1)
            g16 = g16 + jnp.where(iota16 == k, dot, 0.0)
        z = g16 * 0.0625
        g_v[pl.ds(gi * 16, 16)] = 1.0 / (1.0 + jnp.exp(-z))
        return 0

    lax.fori_loop(0, 8, p_body, 0)
    pltpu.sync_copy(g_v, out_hbm.at[pl.ds(wid * 128, 128)])


_final = functools.partial(
    pl.kernel,
    out_type=jax.ShapeDtypeStruct((B,), jnp.float32),
    mesh=_MESH,
    compiler_params=_CPARAMS_NL,
    scratch_types=[
        pltpu.VMEM((128,), jnp.int32),
        pltpu.VMEM((128,), jnp.int32),
        pltpu.VMEM((128, DIM), jnp.float32),
        pltpu.VMEM((128, DIM), jnp.float32),
        pltpu.VMEM((128, DIM), jnp.float32),
        pltpu.VMEM((128,), jnp.float32),
        pltpu.SemaphoreType.DMA,
    ],
)(_final_body)


def kernel(users, items, user_emb, item_emb, edge_index, edge_weight):
    x0 = jnp.concatenate([user_emb, item_emb], axis=0)
    src = edge_index[0].astype(jnp.int32)
    dst = edge_index[1].astype(jnp.int32)
    w = edge_weight.astype(jnp.float32)
    pad = EP - E
    src = jnp.concatenate([src, jnp.zeros((pad,), jnp.int32)]).reshape(EP // 128, 128)
    dst = jnp.concatenate([dst, jnp.full((pad,), NN, jnp.int32)]).reshape(EP // 128, 128)
    w = jnp.concatenate([w, jnp.zeros((pad,), jnp.float32)]).reshape(EP // 128, 128)

    psrc, pdst, pw, cnts = _partition(src, dst, w)

    x1 = _layer(x0, psrc, pdst, pw, cnts)
    x2 = _layer(x1, psrc, pdst, pw, cnts)
    x3 = _layer(x2, psrc, pdst, pw, cnts)

    u1d = users.astype(jnp.int32)
    i1d = items.astype(jnp.int32) + USER_N
    return _final(x0, x1, x2, x3, u1d, i1d)


# speedup vs baseline: 4.9406x; 1.0169x over previous
import functools

import jax
import jax.numpy as jnp
from jax import lax
from jax.experimental import pallas as pl
from jax.experimental.pallas import tpu as pltpu
from jax.experimental.pallas import tpu_sc as plsc

USER_N = 50000
NN = 100000
DIM = 32
E = 1600000
B = 4096
NC = 2
NS = 16
NT = NC * NS
QTR = 25000
NQ = 4
ACC_ROWS = 25088
TRASH = QTR
EP = 1638400
IR_PER_TILE = EP // 128 // NT
SCAN_IR = 8
N_SCAN = IR_PER_TILE // SCAN_IR
CHUNK_E = 1024
PAD_G = CHUNK_E // 16
RCAP_E = 103424
PTOT = NQ * NT * RCAP_E
ZROWS = 784
STRIPE = 1568
WR_LAST = QTR - (NS - 1) * STRIPE

_MESH = plsc.VectorSubcoreMesh(core_axis_name="c", subcore_axis_name="s")
_CPARAMS = pltpu.CompilerParams(use_tc_tiling_on_sc=False)
_CPARAMS_NL = pltpu.CompilerParams(use_tc_tiling_on_sc=False,
                                   needs_layout_passes=False)


def _partition_body(src_hbm, dst_hbm, w_hbm, psrc, pdst, pw, cnts,
                    cs_v, cd_v, cw_v, st_s, st_d, st_w, cnt_v):
    c = lax.axis_index("c")
    s = lax.axis_index("s")
    wid = s * NC + c
    iota16 = lax.iota(jnp.int32, 16)

    def flush(bt, nf):
        base = (bt * NT + wid) * RCAP_E + nf * CHUNK_E
        pltpu.sync_copy(st_s.at[bt].at[pl.ds(0, CHUNK_E)],
                        psrc.at[pl.ds(base, CHUNK_E)])
        pltpu.sync_copy(st_d.at[bt].at[pl.ds(0, CHUNK_E)],
                        pdst.at[pl.ds(base, CHUNK_E)])
        pltpu.sync_copy(st_w.at[bt].at[pl.ds(0, CHUNK_E)],
                        pw.at[pl.ds(base, CHUNK_E)])

    def group_step(carry, d, sg, wg):
        fills, nfs = carry
        bq = ((d >= QTR).astype(jnp.int32)
              + (d >= 2 * QTR).astype(jnp.int32)
              + (d >= 3 * QTR).astype(jnp.int32))
        new_fills, new_nfs = [], []
        for bt in range(NQ):
            mask = bq == bt
            cnt = plsc.all_reduce_population_count(mask)[0]
            fill = fills[bt]
            nf = nfs[bt]
            ld = d - bt * QTR
            plsc.store_compressed(st_s.at[bt].at[pl.ds(fill, 16)], sg,
                                  mask=mask)
            plsc.store_compressed(st_d.at[bt].at[pl.ds(fill, 16)], ld,
                                  mask=mask)
            plsc.store_compressed(st_w.at[bt].at[pl.ds(fill, 16)], wg,
                                  mask=mask)
            fill = fill + cnt
            do_flush = fill >= CHUNK_E

            @pl.when(do_flush)
            def _():
                flush(bt, nf)
                st_s.at[bt][pl.ds(0, 16)] = st_s.at[bt][pl.ds(CHUNK_E, 16)]
                st_d.at[bt][pl.ds(0, 16)] = st_d.at[bt][pl.ds(CHUNK_E, 16)]
                st_w.at[bt][pl.ds(0, 16)] = st_w.at[bt][pl.ds(CHUNK_E, 16)]

            new_fills.append(jnp.where(do_flush, fill - CHUNK_E, fill))
            new_nfs.append(jnp.where(do_flush, nf + 1, nf))
        return (tuple(new_fills), tuple(new_nfs))

    def chunk_body(ci, carry):
        r0 = wid * IR_PER_TILE + ci * SCAN_IR
        pltpu.sync_copy(src_hbm.at[pl.ds(r0, SCAN_IR)], cs_v)
        pltpu.sync_copy(dst_hbm.at[pl.ds(r0, SCAN_IR)], cd_v)
        pltpu.sync_copy(w_hbm.at[pl.ds(r0, SCAN_IR)], cw_v)

        def g_body(gi, carry):
            r = gi // 8
            col = (gi % 8) * 16
            d = cd_v[r, pl.ds(col, 16)]
            sg = cs_v[r, pl.ds(col, 16)]
            wg = cw_v[r, pl.ds(col, 16)]
            return group_step(carry, d, sg, wg)

        return lax.fori_loop(0, SCAN_IR * 8, g_body, carry)

    zero = jnp.int32(0)
    fills, nfs = lax.fori_loop(
        0, N_SCAN, chunk_body,
        ((zero, zero, zero, zero), (zero, zero, zero, zero)))

    t_s = jnp.zeros((16,), jnp.int32)
    t_d = jnp.full((16,), TRASH, jnp.int32)
    t_w = jnp.zeros((16,), jnp.float32)
    cv = jnp.zeros((16,), jnp.int32)
    for bt in range(NQ):
        fill = fills[bt]
        nf = nfs[bt]

        def pad_k(k, _):
            off = fill + 16 * k
            m = (iota16 * 0 + off) < CHUNK_E
            offc = jnp.minimum(off, CHUNK_E)
            plsc.store_compressed(st_s.at[bt].at[pl.ds(offc, 16)], t_s,
                                  mask=m)
            plsc.store_compressed(st_d.at[bt].at[pl.ds(offc, 16)], t_d,
                                  mask=m)
            plsc.store_compressed(st_w.at[bt].at[pl.ds(offc, 16)], t_w,
                                  mask=m)
            return 0

        lax.fori_loop(0, PAD_G, pad_k, 0)

        @pl.when(fill > 0)
        def _():
            flush(bt, nf)

        nf = jnp.where(fill > 0, nf + 1, nf)
        cv = cv + jnp.where(iota16 == bt, nf * CHUNK_E, 0)

    cnt_v[...] = cv
    pltpu.sync_copy(cnt_v, cnts.at[pl.ds(wid * 16, 16)])


_partition = functools.partial(
    pl.kernel,
    out_type=(
        jax.ShapeDtypeStruct((PTOT,), jnp.int32),
        jax.ShapeDtypeStruct((PTOT,), jnp.int32),
        jax.ShapeDtypeStruct((PTOT,), jnp.float32),
        jax.ShapeDtypeStruct((NT * 16,), jnp.int32),
    ),
    mesh=_MESH,
    compiler_params=_CPARAMS_NL,
    scratch_types=[
        pltpu.VMEM((SCAN_IR, 128), jnp.int32),
        pltpu.VMEM((SCAN_IR, 128), jnp.int32),
        pltpu.VMEM((SCAN_IR, 128), jnp.float32),
        pltpu.VMEM((NQ, CHUNK_E + 16), jnp.int32),
        pltpu.VMEM((NQ, CHUNK_E + 16), jnp.int32),
        pltpu.VMEM((NQ, CHUNK_E + 16), jnp.float32),
        pltpu.VMEM((16,), jnp.int32),
    ],
)(_partition_body)


def _layer_body(x_hbm, psrc, pdst, pw, cnts, out_hbm,
                idx_v, pd_v, w_v, rows_v, zeros_v, cnt_v, acc_sh, gsem):
    c = lax.axis_index("c")
    s = lax.axis_index("s")
    iota16 = lax.iota(jnp.int32, 16)

    z16 = jnp.zeros((16,), jnp.float32)

    def z_body(q, _):
        r = q // 2
        col = (q % 2) * 16
        zeros_v[r, pl.ds(col, 16)] = z16
        return 0

    lax.fori_loop(0, ZROWS * 2, z_body, 0, unroll=8)
    zbase = s * STRIPE

    for p in range(2):
        for k in range(2):
            pltpu.sync_copy(zeros_v,
                            acc_sh.at[pl.ds(zbase + k * ZROWS, ZROWS)])
        plsc.subcore_barrier()

        q = p * NC + c
        cbase = q * QTR

        def chunk_body(ci, rbase):
            off = rbase + ci * CHUNK_E
            pltpu.sync_copy(psrc.at[pl.ds(off, CHUNK_E)], idx_v)
            pltpu.sync_copy(pdst.at[pl.ds(off, CHUNK_E)], pd_v)
            pltpu.sync_copy(pw.at[pl.ds(off, CHUNK_E)], w_v)
            cps = [pltpu.async_copy(
                       x_hbm.at[idx_v.at[pl.ds(j * 128, 128)]],
                       rows_v.at[pl.ds(j * 128, 128)], gsem)
                   for j in range(CHUNK_E // 128)]
            for cp in cps:
                cp.wait()

            def e_body(gi, _):
                w16 = w_v[pl.ds(gi * 16, 16)]
                for k in range(16):
                    j = gi * 16 + k
                    wk = w16[k]
                    rows_v[j, pl.ds(0, 16)] = rows_v[j, pl.ds(0, 16)] * wk
                    rows_v[j, pl.ds(16, 16)] = rows_v[j, pl.ds(16, 16)] * wk
                return 0

            lax.fori_loop(0, CHUNK_E // 16, e_body, 0)

            scps = [pltpu.async_copy(
                        rows_v.at[pl.ds(j * 128, 128)],
                        acc_sh.at[pd_v.at[pl.ds(j * 128, 128)]],
                        gsem, add=True)
                    for j in range(CHUNK_E // 128)]
            for cp in scps:
                cp.wait()
            return rbase

        for rg in range(2):
            pt = s * 2 + rg
            pltpu.sync_copy(cnts.at[pl.ds(pt * 16, 16)], cnt_v)
            cvec = cnt_v[...]
            n_edges = jnp.sum(jnp.where(iota16 == q, cvec, 0))
            n_chunks = n_edges // CHUNK_E
            rbase = (q * NT + pt) * RCAP_E
            lax.fori_loop(0, n_chunks, chunk_body, rbase)

        plsc.subcore_barrier()

        @pl.when(s < NS - 1)
        def _():
            pltpu.sync_copy(acc_sh.at[pl.ds(s * STRIPE, STRIPE)],
                            out_hbm.at[pl.ds(cbase + s * STRIPE, STRIPE)])

        @pl.when(s == NS - 1)
        def _():
            pltpu.sync_copy(
                acc_sh.at[pl.ds((NS - 1) * STRIPE, WR_LAST)],
                out_hbm.at[pl.ds(cbase + (NS - 1) * STRIPE, WR_LAST)])


_layer = functools.partial(
    pl.kernel,
    out_type=jax.ShapeDtypeStruct((NN, DIM), jnp.float32),
    mesh=_MESH,
    compiler_params=_CPARAMS_NL,
    scratch_types=[
        pltpu.VMEM((CHUNK_E,), jnp.int32),
        pltpu.VMEM((CHUNK_E,), jnp.int32),
        pltpu.VMEM((CHUNK_E,), jnp.float32),
        pltpu.VMEM((CHUNK_E, DIM), jnp.float32),
        pltpu.VMEM((ZROWS, DIM), jnp.float32),
        pltpu.VMEM((16,), jnp.int32),
        pltpu.VMEM_SHARED((ACC_ROWS, DIM), jnp.float32),
        pltpu.SemaphoreType.DMA,
    ],
)(_layer_body)


def _final_body(x0, x1, x2, x3, u_hbm, i_hbm, out_hbm,
                uidx, iidx, uacc, vacc, tmp, g_v, gsem):
    c = lax.axis_index("c")
    s = lax.axis_index("s")
    wid = s * NC + c

    pltpu.sync_copy(u_hbm.at[pl.ds(wid * 128, 128)], uidx)
    pltpu.sync_copy(i_hbm.at[pl.ds(wid * 128, 128)], iidx)

    def acc_into(dst_ref, first, tabs, idx_ref):
        pltpu.async_copy(first.at[idx_ref], dst_ref, gsem).wait()
        for t in tabs:
            pltpu.async_copy(t.at[idx_ref], tmp, gsem).wait()

            def a_body(qq, _):
                r = qq // 2
                col = (qq % 2) * 16
                dst_ref[r, pl.ds(col, 16)] = (dst_ref[r, pl.ds(col, 16)]
                                              + tmp[r, pl.ds(col, 16)])
                return 0

            lax.fori_loop(0, 256, a_body, 0, unroll=8)

    acc_into(uacc, x0, (x1, x2, x3), uidx)
    acc_into(vacc, x0, (x1, x2, x3), iidx)

    iota16 = lax.iota(jnp.int32, 16)

    def p_body(gi, _):
        g16 = jnp.zeros((16,), jnp.float32)
        for k in range(16):
            j = gi * 16 + k
            u0 = uacc[j, pl.ds(0, 16)]
            u1 = uacc[j, pl.ds(16, 16)]
            v0 = vacc[j, pl.ds(0, 16)]
            v1 = vacc[j, pl.ds(16, 16)]
            dot = jnp.sum(u0 * v0 + u1 * v1)
            g16 = g16 + jnp.where(iota16 == k, dot, 0.0)
        z = g16 * 0.0625
        g_v[pl.ds(gi * 16, 16)] = 1.0 / (1.0 + jnp.exp(-z))
        return 0

    lax.fori_loop(0, 8, p_body, 0)
    pltpu.sync_copy(g_v, out_hbm.at[pl.ds(wid * 128, 128)])


_final = functools.partial(
    pl.kernel,
    out_type=jax.ShapeDtypeStruct((B,), jnp.float32),
    mesh=_MESH,
    compiler_params=_CPARAMS_NL,
    scratch_types=[
        pltpu.VMEM((128,), jnp.int32),
        pltpu.VMEM((128,), jnp.int32),
        pltpu.VMEM((128, DIM), jnp.float32),
        pltpu.VMEM((128, DIM), jnp.float32),
        pltpu.VMEM((128, DIM), jnp.float32),
        pltpu.VMEM((128,), jnp.float32),
        pltpu.SemaphoreType.DMA,
    ],
)(_final_body)


def kernel(users, items, user_emb, item_emb, edge_index, edge_weight):
    x0 = jnp.concatenate([user_emb, item_emb], axis=0)
    src = edge_index[0].astype(jnp.int32)
    dst = edge_index[1].astype(jnp.int32)
    w = edge_weight.astype(jnp.float32)
    pad = EP - E
    src = jnp.concatenate([src, jnp.zeros((pad,), jnp.int32)]).reshape(EP // 128, 128)
    dst = jnp.concatenate([dst, jnp.full((pad,), NN, jnp.int32)]).reshape(EP // 128, 128)
    w = jnp.concatenate([w, jnp.zeros((pad,), jnp.float32)]).reshape(EP // 128, 128)

    psrc, pdst, pw, cnts = _partition(src, dst, w)

    x1 = _layer(x0, psrc, pdst, pw, cnts)
    x2 = _layer(x1, psrc, pdst, pw, cnts)
    x3 = _layer(x2, psrc, pdst, pw, cnts)

    u1d = users.astype(jnp.int32)
    i1d = items.astype(jnp.int32) + USER_N
    return _final(x0, x1, x2, x3, u1d, i1d)
